# baseline (speedup 1.0000x reference)
import functools

import jax
import jax.numpy as jnp
from jax import lax
from jax.experimental import pallas as pl
from jax.experimental.pallas import tpu as pltpu
from jax.experimental.pallas import tpu_sc as plsc

N = 10000
N_PAD = 10240
E = 320000
F = 128
H = 64
G = 64
NC, NS = 2, 16
NW = NC * NS
CHUNK = 128
CPT = 80
NBUF = 2
E_PAD = NW * CPT * CHUNK
RPS = N_PAD // NS
DEG_W = 16

_MESH = dict(core_axis_name="c", subcore_axis_name="s", num_cores=NC,
             num_subcores=NS)


def _sc_deg_body(dst_hbm, zeros_hbm, ones_hbm, out_hbm, dst_v, ones_v, deg_sh):
    c = lax.axis_index("c")
    s = lax.axis_index("s")
    wid = s * NC + c
    base = s * RPS
    pltpu.sync_copy(zeros_hbm.at[pl.ds(base, RPS)],
                    deg_sh.at[pl.ds(base, RPS)])
    pltpu.sync_copy(ones_hbm, ones_v)
    pltpu.sync_copy(dst_hbm.at[wid], dst_v)
    plsc.subcore_barrier()

    def body(j, carry):
        pltpu.sync_copy(ones_v, deg_sh.at[dst_v.at[j]], add=True)
        return carry

    lax.fori_loop(0, CPT, body, 0)
    plsc.subcore_barrier()
    pltpu.sync_copy(deg_sh.at[pl.ds(base, RPS)],
                    out_hbm.at[c, pl.ds(base, RPS)])


def _sc_edge_body(g_hbm, src_hbm, dst_hbm, zeros_hbm, out_hbm,
                  src_v, dst_v, rows_v, gsems, agg_sh):
    c = lax.axis_index("c")
    s = lax.axis_index("s")
    wid = s * NC + c
    base = s * RPS
    pltpu.sync_copy(zeros_hbm.at[pl.ds(base, RPS)],
                    agg_sh.at[pl.ds(base, RPS)])
    pltpu.sync_copy(src_hbm.at[wid], src_v)
    pltpu.sync_copy(dst_hbm.at[wid], dst_v)
    plsc.subcore_barrier()

    def gather(j, b):
        return pltpu.make_async_copy(g_hbm.at[src_v.at[j]], rows_v.at[b],
                                     gsems.at[b])

    gather(0, 0).start()
    gather(1, 1).start()

    def grp_body(grp, carry):
        j0 = grp * 2
        for b in range(2):
            j = j0 + b
            gather(j, b).wait()
            pltpu.sync_copy(rows_v.at[b], agg_sh.at[dst_v.at[j]], add=True)
            gather(j + 2, b).start()
        return carry

    lax.fori_loop(0, CPT // 2 - 1, grp_body, 0)
    for b in range(2):
        j = CPT - 2 + b
        gather(j, b).wait()
        pltpu.sync_copy(rows_v.at[b], agg_sh.at[dst_v.at[j]], add=True)
    plsc.subcore_barrier()
    pltpu.sync_copy(agg_sh.at[pl.ds(base, RPS)],
                    out_hbm.at[c, pl.ds(base, RPS)])


_SC_PARAMS = pltpu.CompilerParams(use_tc_tiling_on_sc=False)


def _sc_deg(dstz, zeros16, ones16):
    return pl.kernel(
        _sc_deg_body,
        out_type=jax.ShapeDtypeStruct((NC, N_PAD, DEG_W), jnp.float32),
        mesh=plsc.VectorSubcoreMesh(**_MESH),
        compiler_params=_SC_PARAMS,
        scratch_types=[
            pltpu.VMEM((CPT, CHUNK), jnp.int32),
            pltpu.VMEM((CHUNK, DEG_W), jnp.float32),
            pltpu.VMEM_SHARED((N_PAD, DEG_W), jnp.float32),
        ],
    )(dstz, zeros16, ones16)


def _sc_edge(g, srcz, dstz, zeros64):
    return pl.kernel(
        _sc_edge_body,
        out_type=jax.ShapeDtypeStruct((NC, N_PAD, H), jnp.float32),
        mesh=plsc.VectorSubcoreMesh(**_MESH),
        compiler_params=_SC_PARAMS,
        scratch_types=[
            pltpu.VMEM((CPT, CHUNK), jnp.int32),
            pltpu.VMEM((CPT, CHUNK), jnp.int32),
            pltpu.VMEM((NBUF, CHUNK, H), jnp.float32),
            pltpu.SemaphoreType.DMA((NBUF,)),
            pltpu.VMEM_SHARED((N_PAD, H), jnp.float32),
        ],
    )(g, srcz, dstz, zeros64)


BLK = 1024
GRID = N_PAD // BLK


def _deg_dis(dp_ref):
    deg = 1.0 + dp_ref[0, :, 0:1] + dp_ref[1, :, 0:1]
    return deg, lax.rsqrt(deg)


def _tc_first_body(x_ref, w_ref, b_ref, dp_ref, g_ref, s_ref):
    deg, dis = _deg_dis(dp_ref)
    h = jnp.dot(x_ref[...], w_ref[...], preferred_element_type=jnp.float32)
    g_ref[...] = h * dis
    s_ref[...] = h / deg + b_ref[...]


def _tc_mid_body(a_ref, s_ref, w_ref, b_ref, dp_ref, g_ref, so_ref):
    deg, dis = _deg_dis(dp_ref)
    xl = jnp.maximum(dis * (a_ref[0] + a_ref[1]) + s_ref[...], 0.0)
    h = jnp.dot(xl, w_ref[...], preferred_element_type=jnp.float32)
    g_ref[...] = h * dis
    so_ref[...] = h / deg + b_ref[...]


def _tc_fin_body(a_ref, s_ref, dp_ref, b3_ref, wl_ref, bl_ref, out_ref,
                 sums, cnt):
    i = pl.program_id(0)
    deg, dis = _deg_dis(dp_ref)
    x4 = jnp.maximum(dis * (a_ref[0] + a_ref[1]) + s_ref[...], 0.0)
    b = b3_ref[0]
    segs = lax.broadcasted_iota(jnp.int32, (G, 1), 0)
    mt = (b == segs).astype(jnp.float32)

    @pl.when(i == 0)
    def _():
        sums[...] = jnp.zeros((G, H), jnp.float32)
        cnt[...] = jnp.zeros((G, H), jnp.float32)

    dot = functools.partial(lax.dot_general,
                            dimension_numbers=(((1,), (0,)), ((), ())),
                            preferred_element_type=jnp.float32)
    sums[...] += dot(mt, x4)
    cnt[...] += dot(mt, jnp.ones((BLK, H), jnp.float32))

    @pl.when(i == pl.num_programs(0) - 1)
    def _():
        pooled = sums[...] / jnp.maximum(cnt[...], 1.0)
        out_ref[...] = jnp.dot(pooled, wl_ref[...],
                               preferred_element_type=jnp.float32) + bl_ref[...]


def _row_spec(width):
    return pl.BlockSpec((BLK, width), lambda i: (i, 0))


def _full_spec(shape):
    nd = len(shape)
    return pl.BlockSpec(shape, lambda i: (0,) * nd)


_PART_SPEC = pl.BlockSpec((NC, BLK, H), lambda i: (0, i, 0))
_DP_SPEC = pl.BlockSpec((2, BLK, DEG_W), lambda i: (0, i, 0))


def _tc_first(x, w1, b1, dp):
    return pl.pallas_call(
        _tc_first_body,
        grid=(GRID,),
        in_specs=[_row_spec(F), _full_spec((F, H)), _full_spec((1, H)),
                  _DP_SPEC],
        out_specs=[_row_spec(H), _row_spec(H)],
        out_shape=[jax.ShapeDtypeStruct((N_PAD, H), jnp.float32),
                   jax.ShapeDtypeStruct((N_PAD, H), jnp.float32)],
    )(x, w1, b1, dp)


def _tc_mid(a, s, w, b, dp):
    return pl.pallas_call(
        _tc_mid_body,
        grid=(GRID,),
        in_specs=[_PART_SPEC, _row_spec(H), _full_spec((H, H)),
                  _full_spec((1, H)), _DP_SPEC],
        out_specs=[_row_spec(H), _row_spec(H)],
        out_shape=[jax.ShapeDtypeStruct((N_PAD, H), jnp.float32),
                   jax.ShapeDtypeStruct((N_PAD, H), jnp.float32)],
    )(a, s, w, b, dp)


def _tc_fin(a, s, dp, batch3, wl, bl):
    return pl.pallas_call(
        _tc_fin_body,
        grid=(GRID,),
        in_specs=[_PART_SPEC, _row_spec(H), _DP_SPEC,
                  pl.BlockSpec((1, 1, BLK), lambda i: (i, 0, 0)),
                  _full_spec((H, 1)), _full_spec((1, 1))],
        out_specs=pl.BlockSpec((G, 1), lambda i: (0, 0)),
        out_shape=jax.ShapeDtypeStruct((G, 1), jnp.float32),
        scratch_shapes=[pltpu.VMEM((G, H), jnp.float32),
                        pltpu.VMEM((G, H), jnp.float32)],
    )(a, s, dp, batch3, wl, bl)


def kernel(x, edge_index, batch, W1, b1, W2, b2, W3, b3, Wl, bl):
    f32 = jnp.float32
    x_pad = jnp.pad(x, ((0, N_PAD - N), (0, 0)))
    pad_e = E_PAD - E
    src = jnp.concatenate(
        [edge_index[0], jnp.full((pad_e,), N, jnp.int32)]).reshape(
            NW, CPT, CHUNK)
    dst = jnp.concatenate(
        [edge_index[1], jnp.full((pad_e,), N, jnp.int32)]).reshape(
            NW, CPT, CHUNK)
    batch3 = jnp.concatenate(
        [batch, jnp.full((N_PAD - N,), -1, jnp.int32)]).reshape(GRID, 1, BLK)
    zeros64 = jnp.zeros((N_PAD, H), f32)
    zeros16 = jnp.zeros((N_PAD, DEG_W), f32)
    ones16 = jnp.ones((CHUNK, DEG_W), f32)

    dp = _sc_deg(dst, zeros16, ones16)
    g1, s1 = _tc_first(x_pad, W1, b1.reshape(1, H), dp)
    a1 = _sc_edge(g1, src, dst, zeros64)
    g2, s2 = _tc_mid(a1, s1, W2, b2.reshape(1, H), dp)
    a2 = _sc_edge(g2, src, dst, zeros64)
    g3, s3 = _tc_mid(a2, s2, W3, b3.reshape(1, H), dp)
    a3 = _sc_edge(g3, src, dst, zeros64)
    return _tc_fin(a3, s3, dp, batch3, Wl, bl.reshape(1, 1))


# speedup vs baseline: 1.0377x; 1.0377x over previous
import functools

import jax
import jax.numpy as jnp
from jax import lax
from jax.experimental import pallas as pl
from jax.experimental.pallas import tpu as pltpu
from jax.experimental.pallas import tpu_sc as plsc

N = 10000
N_PAD = 10240
E = 320000
F = 128
H = 64
G = 64
NC, NS = 2, 16
NW = NC * NS
CHUNK = 128
CPT = 80
NBUF = 2
E_PAD = NW * CPT * CHUNK
RPS = N_PAD // NS
DEG_W = 16

_MESH = dict(core_axis_name="c", subcore_axis_name="s", num_cores=NC,
             num_subcores=NS)


def _sc_deg_body(dst_hbm, zeros_hbm, ones_hbm, out_hbm, dst_v, ones_v, deg_sh):
    c = lax.axis_index("c")
    s = lax.axis_index("s")
    wid = s * NC + c
    base = s * RPS
    pltpu.sync_copy(zeros_hbm.at[pl.ds(base, RPS)],
                    deg_sh.at[pl.ds(base, RPS)])
    pltpu.sync_copy(ones_hbm, ones_v)
    pltpu.sync_copy(dst_hbm.at[wid], dst_v)
    plsc.subcore_barrier()

    def body(j, carry):
        pltpu.sync_copy(ones_v, deg_sh.at[dst_v.at[j]], add=True)
        return carry

    lax.fori_loop(0, CPT, body, 0)
    plsc.subcore_barrier()
    pltpu.sync_copy(deg_sh.at[pl.ds(base, RPS)],
                    out_hbm.at[c, pl.ds(base, RPS)])


def _sc_edge_body(g_hbm, src_hbm, dst_hbm, zeros_hbm, out_hbm,
                  src_v, dst_v, rows0, rows1, sem0, sem1, agg_sh):
    c = lax.axis_index("c")
    s = lax.axis_index("s")
    wid = s * NC + c
    base = s * RPS
    pltpu.sync_copy(zeros_hbm.at[pl.ds(base, RPS)],
                    agg_sh.at[pl.ds(base, RPS)])
    pltpu.sync_copy(src_hbm.at[wid], src_v)
    pltpu.sync_copy(dst_hbm.at[wid], dst_v)
    plsc.subcore_barrier()

    bufs = ((rows0, sem0), (rows1, sem1))

    def gather(j, b):
        rows, sem = bufs[b]
        return pltpu.make_async_copy(g_hbm.at[src_v.at[j]], rows, sem)

    gather(0, 0).start()
    gather(1, 1).start()

    def grp_body(grp, carry):
        j0 = grp * 2
        for b in range(2):
            j = j0 + b
            gather(j, b).wait()
            pltpu.sync_copy(bufs[b][0], agg_sh.at[dst_v.at[j]], add=True)
            gather(j + 2, b).start()
        return carry

    lax.fori_loop(0, CPT // 2 - 1, grp_body, 0)
    for b in range(2):
        j = CPT - 2 + b
        gather(j, b).wait()
        pltpu.sync_copy(bufs[b][0], agg_sh.at[dst_v.at[j]], add=True)
    plsc.subcore_barrier()
    pltpu.sync_copy(agg_sh.at[pl.ds(base, RPS)],
                    out_hbm.at[c, pl.ds(base, RPS)])


_SC_PARAMS = pltpu.CompilerParams(use_tc_tiling_on_sc=False)


def _sc_deg(dstz, zeros16, ones16):
    return pl.kernel(
        _sc_deg_body,
        out_type=jax.ShapeDtypeStruct((NC, N_PAD, DEG_W), jnp.float32),
        mesh=plsc.VectorSubcoreMesh(**_MESH),
        compiler_params=_SC_PARAMS,
        scratch_types=[
            pltpu.VMEM((CPT, CHUNK), jnp.int32),
            pltpu.VMEM((CHUNK, DEG_W), jnp.float32),
            pltpu.VMEM_SHARED((N_PAD, DEG_W), jnp.float32),
        ],
    )(dstz, zeros16, ones16)


def _sc_edge(g, srcz, dstz, zeros64):
    return pl.kernel(
        _sc_edge_body,
        out_type=jax.ShapeDtypeStruct((NC, N_PAD, H), jnp.float32),
        mesh=plsc.VectorSubcoreMesh(**_MESH),
        compiler_params=_SC_PARAMS,
        scratch_types=[
            pltpu.VMEM((CPT, CHUNK), jnp.int32),
            pltpu.VMEM((CPT, CHUNK), jnp.int32),
            pltpu.VMEM((CHUNK, H), jnp.float32),
            pltpu.VMEM((CHUNK, H), jnp.float32),
            pltpu.SemaphoreType.DMA,
            pltpu.SemaphoreType.DMA,
            pltpu.VMEM_SHARED((N_PAD, H), jnp.float32),
        ],
    )(g, srcz, dstz, zeros64)


BLK = 1024
GRID = N_PAD // BLK


def _deg_dis(dp_ref):
    deg = 1.0 + dp_ref[0, :, 0:1] + dp_ref[1, :, 0:1]
    return deg, lax.rsqrt(deg)


def _tc_first_body(x_ref, w_ref, b_ref, dp_ref, g_ref, s_ref):
    deg, dis = _deg_dis(dp_ref)
    h = jnp.dot(x_ref[...], w_ref[...], preferred_element_type=jnp.float32)
    g_ref[...] = h * dis
    s_ref[...] = h / deg + b_ref[...]


def _tc_mid_body(a_ref, s_ref, w_ref, b_ref, dp_ref, g_ref, so_ref):
    deg, dis = _deg_dis(dp_ref)
    xl = jnp.maximum(dis * (a_ref[0] + a_ref[1]) + s_ref[...], 0.0)
    h = jnp.dot(xl, w_ref[...], preferred_element_type=jnp.float32)
    g_ref[...] = h * dis
    so_ref[...] = h / deg + b_ref[...]


def _tc_fin_body(a_ref, s_ref, dp_ref, b3_ref, wl_ref, bl_ref, out_ref,
                 sums, cnt):
    i = pl.program_id(0)
    deg, dis = _deg_dis(dp_ref)
    x4 = jnp.maximum(dis * (a_ref[0] + a_ref[1]) + s_ref[...], 0.0)
    b = b3_ref[0]
    segs = lax.broadcasted_iota(jnp.int32, (G, 1), 0)
    mt = (b == segs).astype(jnp.float32)

    @pl.when(i == 0)
    def _():
        sums[...] = jnp.zeros((G, H), jnp.float32)
        cnt[...] = jnp.zeros((G, H), jnp.float32)

    dot = functools.partial(lax.dot_general,
                            dimension_numbers=(((1,), (0,)), ((), ())),
                            preferred_element_type=jnp.float32)
    sums[...] += dot(mt, x4)
    cnt[...] += dot(mt, jnp.ones((BLK, H), jnp.float32))

    @pl.when(i == pl.num_programs(0) - 1)
    def _():
        pooled = sums[...] / jnp.maximum(cnt[...], 1.0)
        out_ref[...] = jnp.dot(pooled, wl_ref[...],
                               preferred_element_type=jnp.float32) + bl_ref[...]


def _row_spec(width):
    return pl.BlockSpec((BLK, width), lambda i: (i, 0))


def _full_spec(shape):
    nd = len(shape)
    return pl.BlockSpec(shape, lambda i: (0,) * nd)


_PART_SPEC = pl.BlockSpec((NC, BLK, H), lambda i: (0, i, 0))
_DP_SPEC = pl.BlockSpec((2, BLK, DEG_W), lambda i: (0, i, 0))


def _tc_first(x, w1, b1, dp):
    return pl.pallas_call(
        _tc_first_body,
        grid=(GRID,),
        in_specs=[_row_spec(F), _full_spec((F, H)), _full_spec((1, H)),
                  _DP_SPEC],
        out_specs=[_row_spec(H), _row_spec(H)],
        out_shape=[jax.ShapeDtypeStruct((N_PAD, H), jnp.float32),
                   jax.ShapeDtypeStruct((N_PAD, H), jnp.float32)],
    )(x, w1, b1, dp)


def _tc_mid(a, s, w, b, dp):
    return pl.pallas_call(
        _tc_mid_body,
        grid=(GRID,),
        in_specs=[_PART_SPEC, _row_spec(H), _full_spec((H, H)),
                  _full_spec((1, H)), _DP_SPEC],
        out_specs=[_row_spec(H), _row_spec(H)],
        out_shape=[jax.ShapeDtypeStruct((N_PAD, H), jnp.float32),
                   jax.ShapeDtypeStruct((N_PAD, H), jnp.float32)],
    )(a, s, w, b, dp)


def _tc_fin(a, s, dp, batch3, wl, bl):
    return pl.pallas_call(
        _tc_fin_body,
        grid=(GRID,),
        in_specs=[_PART_SPEC, _row_spec(H), _DP_SPEC,
                  pl.BlockSpec((1, 1, BLK), lambda i: (i, 0, 0)),
                  _full_spec((H, 1)), _full_spec((1, 1))],
        out_specs=pl.BlockSpec((G, 1), lambda i: (0, 0)),
        out_shape=jax.ShapeDtypeStruct((G, 1), jnp.float32),
        scratch_shapes=[pltpu.VMEM((G, H), jnp.float32),
                        pltpu.VMEM((G, H), jnp.float32)],
    )(a, s, dp, batch3, wl, bl)


def kernel(x, edge_index, batch, W1, b1, W2, b2, W3, b3, Wl, bl):
    f32 = jnp.float32
    x_pad = jnp.pad(x, ((0, N_PAD - N), (0, 0)))
    pad_e = E_PAD - E
    src = jnp.concatenate(
        [edge_index[0], jnp.full((pad_e,), N, jnp.int32)]).reshape(
            NW, CPT, CHUNK)
    dst = jnp.concatenate(
        [edge_index[1], jnp.full((pad_e,), N, jnp.int32)]).reshape(
            NW, CPT, CHUNK)
    batch3 = jnp.concatenate(
        [batch, jnp.full((N_PAD - N,), -1, jnp.int32)]).reshape(GRID, 1, BLK)
    zeros64 = jnp.zeros((N_PAD, H), f32)
    zeros16 = jnp.zeros((N_PAD, DEG_W), f32)
    ones16 = jnp.ones((CHUNK, DEG_W), f32)

    dp = _sc_deg(dst, zeros16, ones16)
    g1, s1 = _tc_first(x_pad, W1, b1.reshape(1, H), dp)
    a1 = _sc_edge(g1, src, dst, zeros64)
    g2, s2 = _tc_mid(a1, s1, W2, b2.reshape(1, H), dp)
    a2 = _sc_edge(g2, src, dst, zeros64)
    g3, s3 = _tc_mid(a2, s2, W3, b3.reshape(1, H), dp)
    a3 = _sc_edge(g3, src, dst, zeros64)
    return _tc_fin(a3, s3, dp, batch3, Wl, bl.reshape(1, 1))


# speedup vs baseline: 1.4427x; 1.3903x over previous
import functools

import jax
import jax.numpy as jnp
from jax import lax
from jax.experimental import pallas as pl
from jax.experimental.pallas import tpu as pltpu
from jax.experimental.pallas import tpu_sc as plsc

N = 10000
N_PAD = 10240
E = 320000
F = 128
H = 64
G = 64
NC, NS = 2, 16
NW = NC * NS
CHUNK = 128
CPT = 80
NBUF = 2
E_PAD = NW * CPT * CHUNK
RPS = N_PAD // NS
DEG_W = 16
GDT = jnp.bfloat16

_MESH = dict(core_axis_name="c", subcore_axis_name="s", num_cores=NC,
             num_subcores=NS)


def _sc_deg_body(dst_hbm, zeros_hbm, ones_hbm, out_hbm, dst_v, ones_v, deg_sh):
    c = lax.axis_index("c")
    s = lax.axis_index("s")
    wid = s * NC + c
    base = s * RPS
    pltpu.sync_copy(zeros_hbm.at[pl.ds(base, RPS)],
                    deg_sh.at[pl.ds(base, RPS)])
    pltpu.sync_copy(ones_hbm, ones_v)
    pltpu.sync_copy(dst_hbm.at[wid], dst_v)
    plsc.subcore_barrier()

    def body(j, carry):
        pltpu.sync_copy(ones_v, deg_sh.at[dst_v.at[j]], add=True)
        return carry

    lax.fori_loop(0, CPT, body, 0)
    plsc.subcore_barrier()
    pltpu.sync_copy(deg_sh.at[pl.ds(base, RPS)],
                    out_hbm.at[c, pl.ds(base, RPS)])


def _sc_edge_body(g_hbm, src_hbm, dst_hbm, zeros_hbm, out_hbm,
                  src_v, dst_v, rows_v, sem, agg_sh):
    c = lax.axis_index("c")
    s = lax.axis_index("s")
    wid = s * NC + c
    base = s * RPS
    pltpu.sync_copy(zeros_hbm.at[pl.ds(base, RPS)],
                    agg_sh.at[pl.ds(base, RPS)])
    pltpu.sync_copy(src_hbm.at[wid], src_v)
    pltpu.sync_copy(dst_hbm.at[wid], dst_v)
    plsc.subcore_barrier()

    def chunk_body(j, carry):
        pltpu.async_copy(g_hbm.at[src_v.at[j]], rows_v, sem).wait()
        pltpu.sync_copy(rows_v, agg_sh.at[dst_v.at[j]], add=True)
        return carry

    lax.fori_loop(0, CPT, chunk_body, 0)
    plsc.subcore_barrier()
    pltpu.sync_copy(agg_sh.at[pl.ds(base, RPS)],
                    out_hbm.at[c, pl.ds(base, RPS)])


_SC_PARAMS = pltpu.CompilerParams(use_tc_tiling_on_sc=False)


def _sc_deg(dstz, zeros16, ones16):
    return pl.kernel(
        _sc_deg_body,
        out_type=jax.ShapeDtypeStruct((NC, N_PAD, DEG_W), jnp.float32),
        mesh=plsc.VectorSubcoreMesh(**_MESH),
        compiler_params=_SC_PARAMS,
        scratch_types=[
            pltpu.VMEM((CPT, CHUNK), jnp.int32),
            pltpu.VMEM((CHUNK, DEG_W), jnp.float32),
            pltpu.VMEM_SHARED((N_PAD, DEG_W), jnp.float32),
        ],
    )(dstz, zeros16, ones16)


def _sc_edge(g, srcz, dstz, zeros64):
    dt = g.dtype
    return pl.kernel(
        _sc_edge_body,
        out_type=jax.ShapeDtypeStruct((NC, N_PAD, H), dt),
        mesh=plsc.VectorSubcoreMesh(**_MESH),
        compiler_params=_SC_PARAMS,
        scratch_types=[
            pltpu.VMEM((CPT, CHUNK), jnp.int32),
            pltpu.VMEM((CPT, CHUNK), jnp.int32),
            pltpu.VMEM((CHUNK, H), dt),
            pltpu.SemaphoreType.DMA,
            pltpu.VMEM_SHARED((N_PAD, H), dt),
        ],
    )(g, srcz, dstz, zeros64)


BLK = 1024
GRID = N_PAD // BLK


def _deg_dis(dp_ref):
    deg = 1.0 + dp_ref[0, :, 0:1] + dp_ref[1, :, 0:1]
    return deg, lax.rsqrt(deg)


def _tc_first_body(x_ref, w_ref, b_ref, dp_ref, g_ref, s_ref):
    deg, dis = _deg_dis(dp_ref)
    h = jnp.dot(x_ref[...], w_ref[...], preferred_element_type=jnp.float32)
    g_ref[...] = (h * dis).astype(g_ref.dtype)
    s_ref[...] = h / deg + b_ref[...]


def _agg_sum(a_ref):
    return (a_ref[0].astype(jnp.float32) + a_ref[1].astype(jnp.float32))


def _tc_mid_body(a_ref, s_ref, w_ref, b_ref, dp_ref, g_ref, so_ref):
    deg, dis = _deg_dis(dp_ref)
    xl = jnp.maximum(dis * _agg_sum(a_ref) + s_ref[...], 0.0)
    h = jnp.dot(xl, w_ref[...], preferred_element_type=jnp.float32)
    g_ref[...] = (h * dis).astype(g_ref.dtype)
    so_ref[...] = h / deg + b_ref[...]


def _tc_fin_body(a_ref, s_ref, dp_ref, b3_ref, wl_ref, bl_ref, out_ref,
                 sums, cnt):
    i = pl.program_id(0)
    deg, dis = _deg_dis(dp_ref)
    x4 = jnp.maximum(dis * _agg_sum(a_ref) + s_ref[...], 0.0)
    b = b3_ref[0]
    segs = lax.broadcasted_iota(jnp.int32, (G, 1), 0)
    mt = (b == segs).astype(jnp.float32)

    @pl.when(i == 0)
    def _():
        sums[...] = jnp.zeros((G, H), jnp.float32)
        cnt[...] = jnp.zeros((G, H), jnp.float32)

    dot = functools.partial(lax.dot_general,
                            dimension_numbers=(((1,), (0,)), ((), ())),
                            preferred_element_type=jnp.float32)
    sums[...] += dot(mt, x4)
    cnt[...] += dot(mt, jnp.ones((BLK, H), jnp.float32))

    @pl.when(i == pl.num_programs(0) - 1)
    def _():
        pooled = sums[...] / jnp.maximum(cnt[...], 1.0)
        out_ref[...] = jnp.dot(pooled, wl_ref[...],
                               preferred_element_type=jnp.float32) + bl_ref[...]


def _row_spec(width):
    return pl.BlockSpec((BLK, width), lambda i: (i, 0))


def _full_spec(shape):
    nd = len(shape)
    return pl.BlockSpec(shape, lambda i: (0,) * nd)


_PART_SPEC = pl.BlockSpec((NC, BLK, H), lambda i: (0, i, 0))
_DP_SPEC = pl.BlockSpec((2, BLK, DEG_W), lambda i: (0, i, 0))


def _tc_first(x, w1, b1, dp):
    return pl.pallas_call(
        _tc_first_body,
        grid=(GRID,),
        in_specs=[_row_spec(F), _full_spec((F, H)), _full_spec((1, H)),
                  _DP_SPEC],
        out_specs=[_row_spec(H), _row_spec(H)],
        out_shape=[jax.ShapeDtypeStruct((N_PAD, H), GDT),
                   jax.ShapeDtypeStruct((N_PAD, H), jnp.float32)],
    )(x, w1, b1, dp)


def _tc_mid(a, s, w, b, dp):
    return pl.pallas_call(
        _tc_mid_body,
        grid=(GRID,),
        in_specs=[_PART_SPEC, _row_spec(H), _full_spec((H, H)),
                  _full_spec((1, H)), _DP_SPEC],
        out_specs=[_row_spec(H), _row_spec(H)],
        out_shape=[jax.ShapeDtypeStruct((N_PAD, H), GDT),
                   jax.ShapeDtypeStruct((N_PAD, H), jnp.float32)],
    )(a, s, w, b, dp)


def _tc_fin(a, s, dp, batch3, wl, bl):
    return pl.pallas_call(
        _tc_fin_body,
        grid=(GRID,),
        in_specs=[_PART_SPEC, _row_spec(H), _DP_SPEC,
                  pl.BlockSpec((1, 1, BLK), lambda i: (i, 0, 0)),
                  _full_spec((H, 1)), _full_spec((1, 1))],
        out_specs=pl.BlockSpec((G, 1), lambda i: (0, 0)),
        out_shape=jax.ShapeDtypeStruct((G, 1), jnp.float32),
        scratch_shapes=[pltpu.VMEM((G, H), jnp.float32),
                        pltpu.VMEM((G, H), jnp.float32)],
    )(a, s, dp, batch3, wl, bl)


def kernel(x, edge_index, batch, W1, b1, W2, b2, W3, b3, Wl, bl):
    f32 = jnp.float32
    x_pad = jnp.pad(x, ((0, N_PAD - N), (0, 0)))
    pad_e = E_PAD - E
    src = jnp.concatenate(
        [edge_index[0], jnp.full((pad_e,), N, jnp.int32)]).reshape(
            NW, CPT, CHUNK)
    dst = jnp.concatenate(
        [edge_index[1], jnp.full((pad_e,), N, jnp.int32)]).reshape(
            NW, CPT, CHUNK)
    batch3 = jnp.concatenate(
        [batch, jnp.full((N_PAD - N,), -1, jnp.int32)]).reshape(GRID, 1, BLK)
    zeros64 = jnp.zeros((N_PAD, H), GDT)
    zeros16 = jnp.zeros((N_PAD, DEG_W), f32)
    ones16 = jnp.ones((CHUNK, DEG_W), f32)

    dp = _sc_deg(dst, zeros16, ones16)
    g1, s1 = _tc_first(x_pad, W1, b1.reshape(1, H), dp)
    a1 = _sc_edge(g1, src, dst, zeros64)
    g2, s2 = _tc_mid(a1, s1, W2, b2.reshape(1, H), dp)
    a2 = _sc_edge(g2, src, dst, zeros64)
    g3, s3 = _tc_mid(a2, s2, W3, b3.reshape(1, H), dp)
    a3 = _sc_edge(g3, src, dst, zeros64)
    return _tc_fin(a3, s3, dp, batch3, Wl, bl.reshape(1, 1))
